# Initial kernel scaffold; baseline (speedup 1.0000x reference)
import jax
import jax.numpy as jnp
from jax.experimental import pallas as pl


def kernel(x, edge_index, batch, pe_W, pe_b, lin_W1, att1, bias1, posW1, bn_g1, bn_b1, lin_W2, att2, bias2, posW2, bn_g2, bn_b2, fc_W, fc_b):
    raise NotImplementedError("write your pallas kernel here")


# speedup vs baseline: 827.4979x; 827.4979x over previous
import jax
import jax.numpy as jnp
from jax import lax
from jax.experimental import pallas as pl

_NG = 64
_HC = 128
_D = 128


def _fused(x_ref, b_ref, w1_ref, r1_ref, c1_ref, g1_ref, bb1_ref,
           w2_ref, c2_ref, g2_ref, bb2_ref, fc_ref, fcb_ref, out_ref):
    f32 = jnp.float32
    x = x_ref[:]
    b = b_ref[:]
    N = x.shape[0]

    gids = lax.broadcasted_iota(jnp.int32, (N, _NG), 1)
    ohot = (b == gids).astype(f32)
    counts = jnp.sum(ohot, axis=0, keepdims=True)
    ii = lax.broadcasted_iota(jnp.int32, (_NG, _NG), 0)
    jj = lax.broadcasted_iota(jnp.int32, (_NG, _NG), 1)
    tri = (ii < jj).astype(f32)
    starts = jnp.dot(counts, tri, preferred_element_type=f32, precision=lax.Precision.HIGHEST)
    inv_c = 1.0 / jnp.maximum(counts, 1.0)

    tbl = jnp.concatenate([starts, inv_c], axis=0)
    pn = lax.dot_general(ohot, tbl, (((1,), (1,)), ((), ())),
                         preferred_element_type=f32, precision=lax.Precision.HIGHEST)
    rowi = lax.broadcasted_iota(jnp.int32, (N, 1), 0).astype(f32)
    p = (rowi - pn[:, 0:1]) * pn[:, 1:2]

    xt1 = jnp.dot(x, w1_ref[:], preferred_element_type=f32, precision=lax.Precision.HIGHEST)
    xt1 = xt1 + p * r1_ref[:] + c1_ref[:]
    m1 = jnp.mean(xt1, axis=0, keepdims=True)
    d1 = xt1 - m1
    v1 = jnp.mean(d1 * d1, axis=0, keepdims=True)
    h1 = g1_ref[:] * d1 * lax.rsqrt(v1 + 1e-5) + bb1_ref[:]
    h1 = jnp.where(h1 > 0, h1, jnp.exp(h1) - 1.0)

    xt2 = jnp.dot(h1, w2_ref[:], preferred_element_type=f32, precision=lax.Precision.HIGHEST) + c2_ref[:]
    m2 = jnp.mean(xt2, axis=0, keepdims=True)
    d2 = xt2 - m2
    v2 = jnp.mean(d2 * d2, axis=0, keepdims=True)
    h2 = g2_ref[:] * d2 * lax.rsqrt(v2 + 1e-5) + bb2_ref[:]
    h2 = jnp.where(h2 > 0, h2, jnp.exp(h2) - 1.0)

    pooled = lax.dot_general(ohot, h2, (((0,), (0,)), ((), ())),
                             preferred_element_type=f32, precision=lax.Precision.HIGHEST)
    pooled = pooled * jnp.transpose(inv_c)
    out_ref[:] = jnp.dot(pooled, fc_ref[:], preferred_element_type=f32, precision=lax.Precision.HIGHEST) + fcb_ref[:]


def kernel(x, edge_index, batch, pe_W, pe_b, lin_W1, att1, bias1, posW1,
           bn_g1, bn_b1, lin_W2, att2, bias2, posW2, bn_g2, bn_b2, fc_W, fc_b):
    N = x.shape[0]
    W1x_T = lin_W1[:, :_D].T
    W1pe = lin_W1[:, _D:]
    r1 = (W1pe @ (pe_W[:, 0] + pe_W[:, 1]))[None, :]
    c1 = (W1pe @ pe_b + bias1)[None, :]
    out = pl.pallas_call(
        _fused,
        out_shape=jax.ShapeDtypeStruct((_NG, _HC), jnp.float32),
    )(
        x,
        batch.reshape(N, 1),
        W1x_T,
        r1,
        c1,
        bn_g1[None, :],
        bn_b1[None, :],
        lin_W2.T,
        bias2[None, :],
        bn_g2[None, :],
        bn_b2[None, :],
        fc_W.T,
        fc_b[None, :],
    )
    return out


# speedup vs baseline: 1055.5579x; 1.2756x over previous
import jax
import jax.numpy as jnp
from jax import lax
from jax.experimental import pallas as pl

_NG = 64
_HC = 128
_D = 128


def _fused(x_ref, b_ref, w1_ref, r1_ref, c1_ref, g1_ref, bb1_ref,
           w2_ref, c2_ref, g2_ref, bb2_ref, fc_ref, fcb_ref, out_ref):
    f32 = jnp.float32
    x = x_ref[:]
    b = b_ref[:]
    N = x.shape[0]

    gids = lax.broadcasted_iota(jnp.int32, (N, _NG), 1)
    ohot = (b == gids).astype(f32)
    counts = jnp.sum(ohot, axis=0, keepdims=True)
    ii = lax.broadcasted_iota(jnp.int32, (_NG, _NG), 0)
    jj = lax.broadcasted_iota(jnp.int32, (_NG, _NG), 1)
    tri = (ii < jj).astype(f32)
    starts = jnp.dot(counts, tri, preferred_element_type=f32, precision=lax.Precision.HIGHEST)
    inv_c = 1.0 / jnp.maximum(counts, 1.0)

    tbl = jnp.concatenate([starts, inv_c], axis=0)
    pn = lax.dot_general(ohot, tbl, (((1,), (1,)), ((), ())),
                         preferred_element_type=f32, precision=lax.Precision.HIGHEST)
    rowi = lax.broadcasted_iota(jnp.int32, (N, 1), 0).astype(f32)
    p = (rowi - pn[:, 0:1]) * pn[:, 1:2]

    xt1 = jnp.dot(x, w1_ref[:], preferred_element_type=f32)
    xt1 = xt1 + p * r1_ref[:] + c1_ref[:]
    m1 = jnp.mean(xt1, axis=0, keepdims=True)
    d1 = xt1 - m1
    v1 = jnp.mean(d1 * d1, axis=0, keepdims=True)
    h1 = g1_ref[:] * d1 * lax.rsqrt(v1 + 1e-5) + bb1_ref[:]
    h1 = jnp.where(h1 > 0, h1, jnp.exp(h1) - 1.0)

    xt2 = jnp.dot(h1, w2_ref[:], preferred_element_type=f32) + c2_ref[:]
    m2 = jnp.mean(xt2, axis=0, keepdims=True)
    d2 = xt2 - m2
    v2 = jnp.mean(d2 * d2, axis=0, keepdims=True)
    h2 = g2_ref[:] * d2 * lax.rsqrt(v2 + 1e-5) + bb2_ref[:]
    h2 = jnp.where(h2 > 0, h2, jnp.exp(h2) - 1.0)

    pooled = lax.dot_general(ohot, h2, (((0,), (0,)), ((), ())),
                             preferred_element_type=f32, precision=lax.Precision.HIGHEST)
    pooled = pooled * jnp.transpose(inv_c)
    out_ref[:] = jnp.dot(pooled, fc_ref[:], preferred_element_type=f32, precision=lax.Precision.HIGHEST) + fcb_ref[:]


def kernel(x, edge_index, batch, pe_W, pe_b, lin_W1, att1, bias1, posW1,
           bn_g1, bn_b1, lin_W2, att2, bias2, posW2, bn_g2, bn_b2, fc_W, fc_b):
    N = x.shape[0]
    W1x_T = lin_W1[:, :_D].T
    W1pe = lin_W1[:, _D:]
    r1 = (W1pe @ (pe_W[:, 0] + pe_W[:, 1]))[None, :]
    c1 = (W1pe @ pe_b + bias1)[None, :]
    out = pl.pallas_call(
        _fused,
        out_shape=jax.ShapeDtypeStruct((_NG, _HC), jnp.float32),
    )(
        x,
        batch.reshape(N, 1),
        W1x_T,
        r1,
        c1,
        bn_g1[None, :],
        bn_b1[None, :],
        lin_W2.T,
        bias2[None, :],
        bn_g2[None, :],
        bn_b2[None, :],
        fc_W.T,
        fc_b[None, :],
    )
    return out


# speedup vs baseline: 1247.1572x; 1.1815x over previous
import jax
import jax.numpy as jnp
from jax import lax
from jax.experimental import pallas as pl

_NG = 64
_HC = 128
_D = 128


def _fused(x_ref, b_ref, w1_ref, r1_ref, c1_ref, g1_ref, bb1_ref,
           w2_ref, c2_ref, g2_ref, bb2_ref, fc_ref, fcb_ref, out_ref):
    f32 = jnp.float32
    x = x_ref[:]
    b = b_ref[:]
    N = x.shape[0]

    gids = lax.broadcasted_iota(jnp.int32, (N, _NG), 1)
    ohot = (b == gids).astype(f32)
    counts = jnp.sum(ohot, axis=0, keepdims=True)
    ii = lax.broadcasted_iota(jnp.int32, (_NG, _NG), 0)
    jj = lax.broadcasted_iota(jnp.int32, (_NG, _NG), 1)
    tri = (ii < jj).astype(f32)
    starts = jnp.dot(counts, tri, preferred_element_type=f32, precision=lax.Precision.HIGHEST)
    inv_c = 1.0 / jnp.maximum(counts, 1.0)

    tbl = jnp.concatenate([starts, inv_c], axis=0)
    pn = lax.dot_general(ohot, tbl, (((1,), (1,)), ((), ())),
                         preferred_element_type=f32, precision=lax.Precision.HIGHEST)
    rowi = lax.broadcasted_iota(jnp.int32, (N, 1), 0).astype(f32)
    p = (rowi - pn[:, 0:1]) * pn[:, 1:2]

    inv_n = 1.0 / N
    ones_row = jnp.ones((1, N), f32)

    xt1 = jnp.dot(x, w1_ref[:], preferred_element_type=f32)
    xt1 = xt1 + p * r1_ref[:] + c1_ref[:]
    m1 = jnp.dot(ones_row, xt1, preferred_element_type=f32) * inv_n
    s1 = jnp.dot(ones_row, xt1 * xt1, preferred_element_type=f32) * inv_n
    v1 = s1 - m1 * m1
    h1 = g1_ref[:] * (xt1 - m1) * lax.rsqrt(v1 + 1e-5) + bb1_ref[:]
    h1 = jnp.where(h1 > 0, h1, jnp.exp(h1) - 1.0)

    xt2 = jnp.dot(h1, w2_ref[:], preferred_element_type=f32) + c2_ref[:]
    m2 = jnp.dot(ones_row, xt2, preferred_element_type=f32) * inv_n
    s2 = jnp.dot(ones_row, xt2 * xt2, preferred_element_type=f32) * inv_n
    v2 = s2 - m2 * m2
    h2 = g2_ref[:] * (xt2 - m2) * lax.rsqrt(v2 + 1e-5) + bb2_ref[:]
    h2 = jnp.where(h2 > 0, h2, jnp.exp(h2) - 1.0)

    pooled = lax.dot_general(ohot, h2, (((0,), (0,)), ((), ())),
                             preferred_element_type=f32)
    pooled = pooled * jnp.transpose(inv_c)
    out_ref[:] = jnp.dot(pooled, fc_ref[:], preferred_element_type=f32, precision=lax.Precision.HIGHEST) + fcb_ref[:]


def kernel(x, edge_index, batch, pe_W, pe_b, lin_W1, att1, bias1, posW1,
           bn_g1, bn_b1, lin_W2, att2, bias2, posW2, bn_g2, bn_b2, fc_W, fc_b):
    N = x.shape[0]
    W1x_T = lin_W1[:, :_D].T
    W1pe = lin_W1[:, _D:]
    r1 = (W1pe @ (pe_W[:, 0] + pe_W[:, 1]))[None, :]
    c1 = (W1pe @ pe_b + bias1)[None, :]
    out = pl.pallas_call(
        _fused,
        out_shape=jax.ShapeDtypeStruct((_NG, _HC), jnp.float32),
    )(
        x,
        batch.reshape(N, 1),
        W1x_T,
        r1,
        c1,
        bn_g1[None, :],
        bn_b1[None, :],
        lin_W2.T,
        bias2[None, :],
        bn_g2[None, :],
        bn_b2[None, :],
        fc_W.T,
        fc_b[None, :],
    )
    return out


# speedup vs baseline: 1532.4721x; 1.2288x over previous
import jax
import jax.numpy as jnp
from jax import lax
from jax.experimental import pallas as pl

_NG = 64
_HC = 128
_D = 128


def _fused(x_ref, b_ref, w1_ref, r1_ref, c1_ref, g1_ref, bb1_ref,
           w2_ref, c2_ref, g2_ref, bb2_ref, fc_ref, fcb_ref, out_ref):
    f32 = jnp.float32
    x = x_ref[:]
    b = b_ref[:]
    N = x.shape[0]

    gids = lax.broadcasted_iota(jnp.int32, (N, _NG), 1)
    ohot = (b == gids).astype(f32)
    counts = jnp.sum(ohot, axis=0, keepdims=True)
    ii = lax.broadcasted_iota(jnp.int32, (_NG, _NG), 0)
    jj = lax.broadcasted_iota(jnp.int32, (_NG, _NG), 1)
    tri = (ii < jj).astype(f32)
    starts = jnp.dot(counts, tri, preferred_element_type=f32, precision=lax.Precision.HIGHEST)
    inv_c = 1.0 / jnp.maximum(counts, 1.0)

    s_div = jnp.floor(starts * (1.0 / 128.0))
    s_mod = starts - 128.0 * s_div
    tbl = jnp.concatenate([s_div, s_mod, inv_c], axis=0)
    pn = lax.dot_general(ohot, tbl, (((1,), (1,)), ((), ())),
                         preferred_element_type=f32)
    rowi = lax.broadcasted_iota(jnp.int32, (N, 1), 0).astype(f32)
    p = (rowi - 128.0 * pn[:, 0:1] - pn[:, 1:2]) * pn[:, 2:3]

    inv_n = 1.0 / N
    ones_row = jnp.ones((1, N), f32)

    xt1 = jnp.dot(x, w1_ref[:], preferred_element_type=f32)
    xt1 = xt1 + p * r1_ref[:] + c1_ref[:]
    m1 = jnp.dot(ones_row, xt1, preferred_element_type=f32) * inv_n
    s1 = jnp.dot(ones_row, xt1 * xt1, preferred_element_type=f32) * inv_n
    v1 = s1 - m1 * m1
    h1 = g1_ref[:] * (xt1 - m1) * lax.rsqrt(v1 + 1e-5) + bb1_ref[:]
    h1 = jnp.where(h1 > 0, h1, jnp.exp(h1) - 1.0)

    xt2 = jnp.dot(h1, w2_ref[:], preferred_element_type=f32) + c2_ref[:]
    m2 = jnp.dot(ones_row, xt2, preferred_element_type=f32) * inv_n
    s2 = jnp.dot(ones_row, xt2 * xt2, preferred_element_type=f32) * inv_n
    v2 = s2 - m2 * m2
    h2 = g2_ref[:] * (xt2 - m2) * lax.rsqrt(v2 + 1e-5) + bb2_ref[:]
    h2 = jnp.where(h2 > 0, h2, jnp.exp(h2) - 1.0)

    pooled = lax.dot_general(ohot, h2, (((0,), (0,)), ((), ())),
                             preferred_element_type=f32)
    pooled = pooled * jnp.transpose(inv_c)
    out_ref[:] = jnp.dot(pooled, fc_ref[:], preferred_element_type=f32, precision=lax.Precision.HIGHEST) + fcb_ref[:]


def kernel(x, edge_index, batch, pe_W, pe_b, lin_W1, att1, bias1, posW1,
           bn_g1, bn_b1, lin_W2, att2, bias2, posW2, bn_g2, bn_b2, fc_W, fc_b):
    N = x.shape[0]
    W1x_T = lin_W1[:, :_D].T
    W1pe = lin_W1[:, _D:]
    r1 = (W1pe @ (pe_W[:, 0] + pe_W[:, 1]))[None, :]
    c1 = (W1pe @ pe_b + bias1)[None, :]
    out = pl.pallas_call(
        _fused,
        out_shape=jax.ShapeDtypeStruct((_NG, _HC), jnp.float32),
    )(
        x,
        batch.reshape(N, 1),
        W1x_T,
        r1,
        c1,
        bn_g1[None, :],
        bn_b1[None, :],
        lin_W2.T,
        bias2[None, :],
        bn_g2[None, :],
        bn_b2[None, :],
        fc_W.T,
        fc_b[None, :],
    )
    return out
